# baseline (speedup 1.0000x reference)
import jax
import jax.numpy as jnp
from jax import lax
from jax.experimental import pallas as pl
from jax.experimental.pallas import tpu as pltpu
from jax.experimental.pallas import tpu_sc as plsc

B = 4
N = 100000
Q = 128
D = 256
L = 16
NC = 2
NS = 16
WPB = 8
NSH = N // WPB
UNROLL = 4
CH = -(-NSH // (L * UNROLL)) * UNROLL
NPAD = CH * L


def _splat(x, dtype=None):
    v = jnp.broadcast_to(x, (L,))
    return v if dtype is None else v.astype(dtype)


def _fps_body(xs, ys, zs, qw, q_out, samp_out,
              x_v, y_v, z_v, d_v, qrows_v, samp_v, cand_v, bc_v, cands_v,
              cand_sh, bc_sh):
    c = lax.axis_index("c")
    s = lax.axis_index("s")
    g = s // WPB
    j = s % WPB
    b = c * (NS // WPB) + g
    r = b * WPB + j

    lane = lax.broadcasted_iota(jnp.int32, (L,), 0)
    ninf16 = jnp.full((L,), -jnp.inf, jnp.float32)
    lane0 = lane == 0
    rt0 = jnp.minimum(s, 0)

    def _idx(k):
        return _splat(rt0 + k, jnp.int32)

    def _lanek(v, k):
        return jnp.max(jnp.where(lane == k, v, ninf16))

    pltpu.sync_copy(qw.at[pl.ds(j * 16, 16)], qrows_v)
    pltpu.sync_copy(qrows_v, q_out.at[b, pl.ds(j * 16, 16)])

    pltpu.sync_copy(xs.at[r], x_v)
    pltpu.sync_copy(ys.at[r], y_v)
    pltpu.sync_copy(zs.at[r], z_v)

    def init_chunk(ci, _):
        gi = ci * L + lane
        d_v[pl.ds(ci * L, L)] = jnp.where(gi < NSH, jnp.inf, -jnp.inf).astype(jnp.float32)
        return 0

    lax.fori_loop(0, CH, init_chunk, 0)

    @pl.when(j == 0)
    def _():
        x0 = _splat(_lanek(x_v[pl.ds(0, L)], 0))
        y0 = _splat(_lanek(y_v[pl.ds(0, L)], 0))
        z0 = _splat(_lanek(z_v[pl.ds(0, L)], 0))
        plsc.store_scatter(samp_v, [_idx(0), _idx(0)], x0, mask=lane0)
        plsc.store_scatter(samp_v, [_idx(0), _idx(1)], y0, mask=lane0)
        plsc.store_scatter(samp_v, [_idx(0), _idx(2)], z0, mask=lane0)
        row = jnp.where(lane == 0, x0,
              jnp.where(lane == 1, y0,
              jnp.where(lane == 2, z0, jnp.float32(0.0))))
        cand_v[...] = row
        pltpu.sync_copy(cand_v, bc_sh.at[c, g])

    plsc.subcore_barrier()
    pltpu.sync_copy(bc_sh.at[c, g], bc_v)
    brow0 = bc_v[...]
    lastx = _splat(_lanek(brow0, 0))
    lasty = _splat(_lanek(brow0, 1))
    lastz = _splat(_lanek(brow0, 2))

    shard_base = j * NSH

    def step(t, carry):
        lx, ly, lz = carry

        def chunk(cb, st):
            bestv, bestc = st
            for u in range(UNROLL):
                ci = cb * UNROLL + u
                sl = pl.ds(ci * L, L)
                dx = x_v[sl] - lx
                dy = y_v[sl] - ly
                dz = z_v[sl] - lz
                dd = dx * dx + dy * dy + dz * dz
                nd = jnp.minimum(d_v[sl], dd)
                d_v[sl] = nd
                gt = nd > bestv
                bestv = jnp.where(gt, nd, bestv)
                bestc = jnp.where(gt, _splat(ci, jnp.int32), bestc)
            return bestv, bestc

        bestv, bestc = lax.fori_loop(0, CH // UNROLL, chunk,
                                     (ninf16, _splat(rt0, jnp.int32)))

        m = jnp.max(bestv)
        gvec = shard_base + bestc * L + lane
        eq = bestv == _splat(m)
        lg = jnp.min(jnp.where(eq, gvec, jnp.int32(2**31 - 1)))
        p = _splat(lg - shard_base, jnp.int32)
        wx = plsc.load_gather(x_v, [p])
        wy = plsc.load_gather(y_v, [p])
        wz = plsc.load_gather(z_v, [p])

        rowv = jnp.where(lane == 0, _splat(m),
               jnp.where(lane == 1, wx,
               jnp.where(lane == 2, wy,
               jnp.where(lane == 3, wz,
               _splat(lg.astype(jnp.float32))))))
        cand_v[...] = rowv
        pltpu.sync_copy(cand_v, cand_sh.at[c, g, j])
        plsc.subcore_barrier()

        @pl.when(j == 0)
        def _():
            pltpu.sync_copy(cand_sh.at[c, g], cands_v)
            row0 = cands_v[0]
            bv = _lanek(row0, 0)
            bx = _lanek(row0, 1)
            by = _lanek(row0, 2)
            bz = _lanek(row0, 3)
            bi = _lanek(row0, 4)
            for jj in range(1, WPB):
                rw = cands_v[jj]
                vj = _lanek(rw, 0)
                xj = _lanek(rw, 1)
                yj = _lanek(rw, 2)
                zj = _lanek(rw, 3)
                ij = _lanek(rw, 4)
                take = (vj > bv) | ((vj == bv) & (ij < bi))
                bv = jnp.where(take, vj, bv)
                bx = jnp.where(take, xj, bx)
                by = jnp.where(take, yj, by)
                bz = jnp.where(take, zj, bz)
                bi = jnp.where(take, ij, bi)
            wxx = _splat(bx)
            wyy = _splat(by)
            wzz = _splat(bz)
            tv = _splat(t, jnp.int32)
            plsc.store_scatter(samp_v, [tv, _idx(0)], wxx, mask=lane0)
            plsc.store_scatter(samp_v, [tv, _idx(1)], wyy, mask=lane0)
            plsc.store_scatter(samp_v, [tv, _idx(2)], wzz, mask=lane0)
            brow = jnp.where(lane == 0, wxx,
                   jnp.where(lane == 1, wyy,
                   jnp.where(lane == 2, wzz, jnp.float32(0.0))))
            cand_v[...] = brow
            pltpu.sync_copy(cand_v, bc_sh.at[c, g])

        plsc.subcore_barrier()
        pltpu.sync_copy(bc_sh.at[c, g], bc_v)
        nrow = bc_v[...]
        nlx = _splat(_lanek(nrow, 0))
        nly = _splat(_lanek(nrow, 1))
        nlz = _splat(_lanek(nrow, 2))
        return nlx, nly, nlz

    lax.fori_loop(1, Q, step, (lastx, lasty, lastz))

    @pl.when(j == 0)
    def _():
        pltpu.sync_copy(samp_v, samp_out.at[b])


_fps_call = pl.kernel(
    _fps_body,
    out_type=(
        jax.ShapeDtypeStruct((B, Q, D), jnp.float32),
        jax.ShapeDtypeStruct((B, Q, 3), jnp.float32),
    ),
    mesh=plsc.VectorSubcoreMesh(core_axis_name="c", subcore_axis_name="s",
                                num_cores=NC, num_subcores=NS),
    compiler_params=pltpu.CompilerParams(needs_layout_passes=False),
    scratch_types=[
        pltpu.VMEM((NPAD,), jnp.float32),
        pltpu.VMEM((NPAD,), jnp.float32),
        pltpu.VMEM((NPAD,), jnp.float32),
        pltpu.VMEM((NPAD,), jnp.float32),
        pltpu.VMEM((16, D), jnp.float32),
        pltpu.VMEM((Q, 3), jnp.float32),
        pltpu.VMEM((L,), jnp.float32),
        pltpu.VMEM((L,), jnp.float32),
        pltpu.VMEM((WPB, L), jnp.float32),
        pltpu.HBM((NC, NS // WPB, WPB, L), jnp.float32),
        pltpu.HBM((NC, NS // WPB, L), jnp.float32),
    ],
)


def kernel(coordinates, query_weight):
    pts = coordinates.reshape(B, WPB, NSH, 3)
    pts = jnp.pad(pts, ((0, 0), (0, 0), (0, NPAD - NSH), (0, 0)))
    xs = pts[..., 0].reshape(B * WPB, NPAD)
    ys = pts[..., 1].reshape(B * WPB, NPAD)
    zs = pts[..., 2].reshape(B * WPB, NPAD)
    queries, sampled = _fps_call(xs, ys, zs, query_weight)
    return (queries, sampled)


# speedup vs baseline: 1.0561x; 1.0561x over previous
import jax
import jax.numpy as jnp
from jax import lax
from jax.experimental import pallas as pl
from jax.experimental.pallas import tpu as pltpu
from jax.experimental.pallas import tpu_sc as plsc

B = 4
N = 100000
Q = 128
D = 256
L = 16
NC = 2
NS = 16
WPB = 8
NSH = N // WPB
UNROLL = 4
CH = -(-NSH // (L * UNROLL)) * UNROLL
NPAD = CH * L


def _splat(x, dtype=None):
    v = jnp.broadcast_to(x, (L,))
    return v if dtype is None else v.astype(dtype)


def _fps_body(xs, ys, zs, qw, q_out, samp_out,
              x_v, y_v, z_v, d_v, qrows_v, samp_v, cand_v, bc_v, cands_v,
              cand_sh, bc_sh):
    c = lax.axis_index("c")
    s = lax.axis_index("s")
    g = s // WPB
    j = s % WPB
    b = c * (NS // WPB) + g
    r = b * WPB + j

    lane = lax.broadcasted_iota(jnp.int32, (L,), 0)
    ninf16 = jnp.full((L,), -jnp.inf, jnp.float32)
    lane0 = lane == 0
    rt0 = jnp.minimum(s, 0)

    def _idx(k):
        return _splat(rt0 + k, jnp.int32)

    def _lanek(v, k):
        return jnp.max(jnp.where(lane == k, v, ninf16))

    pltpu.sync_copy(qw.at[pl.ds(j * 16, 16)], qrows_v)
    pltpu.sync_copy(qrows_v, q_out.at[b, pl.ds(j * 16, 16)])

    pltpu.sync_copy(xs.at[r], x_v)
    pltpu.sync_copy(ys.at[r], y_v)
    pltpu.sync_copy(zs.at[r], z_v)

    def init_chunk(ci, _):
        gi = ci * L + lane
        d_v[pl.ds(ci * L, L)] = jnp.where(gi < NSH, jnp.inf, -jnp.inf).astype(jnp.float32)
        return 0

    lax.fori_loop(0, CH, init_chunk, 0)

    @pl.when(j == 0)
    def _():
        x0 = _splat(_lanek(x_v[pl.ds(0, L)], 0))
        y0 = _splat(_lanek(y_v[pl.ds(0, L)], 0))
        z0 = _splat(_lanek(z_v[pl.ds(0, L)], 0))
        plsc.store_scatter(samp_v, [_idx(0), _idx(0)], x0, mask=lane0)
        plsc.store_scatter(samp_v, [_idx(0), _idx(1)], y0, mask=lane0)
        plsc.store_scatter(samp_v, [_idx(0), _idx(2)], z0, mask=lane0)
        row = jnp.where(lane == 0, x0,
              jnp.where(lane == 1, y0,
              jnp.where(lane == 2, z0, jnp.float32(0.0))))
        cand_v[...] = row
        pltpu.sync_copy(cand_v, bc_sh.at[c, g])

    plsc.subcore_barrier()
    pltpu.sync_copy(bc_sh.at[c, g], bc_v)
    brow0 = bc_v[...]
    lastx = _splat(_lanek(brow0, 0))
    lasty = _splat(_lanek(brow0, 1))
    lastz = _splat(_lanek(brow0, 2))

    shard_base = j * NSH

    BIGIDX = jnp.float32(3.0e9)

    def step(t, carry):
        lx, ly, lz = carry

        def chunk(cb, st):
            vs = list(st)
            for u in range(UNROLL):
                ci = cb * UNROLL + u
                sl = pl.ds(ci * L, L)
                dx = x_v[sl] - lx
                dy = y_v[sl] - ly
                dz = z_v[sl] - lz
                dd = dx * dx + dy * dy + dz * dz
                nd = jnp.minimum(d_v[sl], dd)
                d_v[sl] = nd
                gt = nd > vs[2 * u]
                vs[2 * u] = jnp.where(gt, nd, vs[2 * u])
                vs[2 * u + 1] = jnp.where(gt, _splat(ci, jnp.int32), vs[2 * u + 1])
            return tuple(vs)

        init = []
        for _ in range(UNROLL):
            init += [ninf16, _splat(rt0, jnp.int32)]
        st = lax.fori_loop(0, CH // UNROLL, chunk, tuple(init))
        bestv, bestc = st[0], st[1]
        for u in range(1, UNROLL):
            vu, cu = st[2 * u], st[2 * u + 1]
            tk = (vu > bestv) | ((vu == bestv) & (cu < bestc))
            bestv = jnp.where(tk, vu, bestv)
            bestc = jnp.where(tk, cu, bestc)

        m = jnp.max(bestv)
        gvec = shard_base + bestc * L + lane
        eq = bestv == _splat(m)
        lg = jnp.min(jnp.where(eq, gvec, jnp.int32(2**31 - 1)))
        p = _splat(lg - shard_base, jnp.int32)
        wx = plsc.load_gather(x_v, [p])
        wy = plsc.load_gather(y_v, [p])
        wz = plsc.load_gather(z_v, [p])

        rowv = jnp.where(lane == 0, _splat(m),
               jnp.where(lane == 1, wx,
               jnp.where(lane == 2, wy,
               jnp.where(lane == 3, wz,
               _splat(lg.astype(jnp.float32))))))
        cand_v[...] = rowv
        par = jnp.bitwise_and(t, 1)
        pltpu.sync_copy(cand_v, cand_sh.at[c, g, par, j])
        plsc.subcore_barrier()

        pltpu.sync_copy(cand_sh.at[c, g, par], cands_v)
        rowl = jnp.minimum(lane, WPB - 1)
        valid = lane < WPB
        vals = jnp.where(valid, plsc.load_gather(cands_v, [rowl, _idx(0)]), ninf16)
        gxs = plsc.load_gather(cands_v, [rowl, _idx(1)])
        gys = plsc.load_gather(cands_v, [rowl, _idx(2)])
        gzs = plsc.load_gather(cands_v, [rowl, _idx(3)])
        gis = jnp.where(valid, plsc.load_gather(cands_v, [rowl, _idx(4)]), _splat(BIGIDX))
        mm = jnp.max(vals)
        eqw = (vals == _splat(mm)) & valid
        gg = jnp.min(jnp.where(eqw, gis, _splat(BIGIDX)))
        eqg = eqw & (gis == _splat(gg))
        wxx = _splat(jnp.max(jnp.where(eqg, gxs, ninf16)))
        wyy = _splat(jnp.max(jnp.where(eqg, gys, ninf16)))
        wzz = _splat(jnp.max(jnp.where(eqg, gzs, ninf16)))

        @pl.when(j == 0)
        def _():
            tv = _splat(t, jnp.int32)
            plsc.store_scatter(samp_v, [tv, _idx(0)], wxx, mask=lane0)
            plsc.store_scatter(samp_v, [tv, _idx(1)], wyy, mask=lane0)
            plsc.store_scatter(samp_v, [tv, _idx(2)], wzz, mask=lane0)

        return wxx, wyy, wzz

    lax.fori_loop(1, Q, step, (lastx, lasty, lastz))

    @pl.when(j == 0)
    def _():
        pltpu.sync_copy(samp_v, samp_out.at[b])


_fps_call = pl.kernel(
    _fps_body,
    out_type=(
        jax.ShapeDtypeStruct((B, Q, D), jnp.float32),
        jax.ShapeDtypeStruct((B, Q, 3), jnp.float32),
    ),
    mesh=plsc.VectorSubcoreMesh(core_axis_name="c", subcore_axis_name="s",
                                num_cores=NC, num_subcores=NS),
    compiler_params=pltpu.CompilerParams(needs_layout_passes=False),
    scratch_types=[
        pltpu.VMEM((NPAD,), jnp.float32),
        pltpu.VMEM((NPAD,), jnp.float32),
        pltpu.VMEM((NPAD,), jnp.float32),
        pltpu.VMEM((NPAD,), jnp.float32),
        pltpu.VMEM((16, D), jnp.float32),
        pltpu.VMEM((Q, 3), jnp.float32),
        pltpu.VMEM((L,), jnp.float32),
        pltpu.VMEM((L,), jnp.float32),
        pltpu.VMEM((WPB, L), jnp.float32),
        pltpu.HBM((NC, NS // WPB, 2, WPB, L), jnp.float32),
        pltpu.HBM((NC, NS // WPB, L), jnp.float32),
    ],
)


def kernel(coordinates, query_weight):
    pts = coordinates.reshape(B, WPB, NSH, 3)
    pts = jnp.pad(pts, ((0, 0), (0, 0), (0, NPAD - NSH), (0, 0)))
    xs = pts[..., 0].reshape(B * WPB, NPAD)
    ys = pts[..., 1].reshape(B * WPB, NPAD)
    zs = pts[..., 2].reshape(B * WPB, NPAD)
    queries, sampled = _fps_call(xs, ys, zs, query_weight)
    return (queries, sampled)


# speedup vs baseline: 2.7029x; 2.5592x over previous
import jax
import jax.numpy as jnp
from jax import lax
from jax.experimental import pallas as pl
from jax.experimental.pallas import tpu as pltpu
from jax.experimental.pallas import tpu_sc as plsc

B = 4
N = 100000
Q = 128
D = 256
L = 16
NC = 2
NS = 16
WPB = 8
NSH = N // WPB
UNROLL = 4
CH = -(-NSH // (L * UNROLL)) * UNROLL
NPAD = CH * L


def _splat(x, dtype=None):
    v = jnp.broadcast_to(x, (L,))
    return v if dtype is None else v.astype(dtype)


def _fps_body(xs, ys, zs, qw, q_out, samp_out,
              x_v, y_v, z_v, d_v, qrows_v, samp_v, cand_v, bc_v, cands_v,
              cand_sh, bc_sh):
    c = lax.axis_index("c")
    s = lax.axis_index("s")
    g = s // WPB
    j = s % WPB
    b = c * (NS // WPB) + g
    r = b * WPB + j

    lane = lax.broadcasted_iota(jnp.int32, (L,), 0)
    ninf16 = jnp.full((L,), -jnp.inf, jnp.float32)
    lane0 = lane == 0
    rt0 = jnp.minimum(s, 0)

    def _idx(k):
        return _splat(rt0 + k, jnp.int32)

    def _lanek(v, k):
        return jnp.max(jnp.where(lane == k, v, ninf16))

    pltpu.sync_copy(qw.at[pl.ds(j * 16, 16)], qrows_v)
    pltpu.sync_copy(qrows_v, q_out.at[b, pl.ds(j * 16, 16)])

    pltpu.sync_copy(xs.at[r], x_v)
    pltpu.sync_copy(ys.at[r], y_v)
    pltpu.sync_copy(zs.at[r], z_v)

    def init_chunk(ci, _):
        gi = ci * L + lane
        d_v[pl.ds(ci * L, L)] = jnp.where(gi < NSH, jnp.inf, -jnp.inf).astype(jnp.float32)
        return 0

    lax.fori_loop(0, CH, init_chunk, 0)

    @pl.when(j == 0)
    def _():
        x0 = _splat(_lanek(x_v[pl.ds(0, L)], 0))
        y0 = _splat(_lanek(y_v[pl.ds(0, L)], 0))
        z0 = _splat(_lanek(z_v[pl.ds(0, L)], 0))
        plsc.store_scatter(samp_v, [_idx(0), _idx(0)], x0, mask=lane0)
        plsc.store_scatter(samp_v, [_idx(0), _idx(1)], y0, mask=lane0)
        plsc.store_scatter(samp_v, [_idx(0), _idx(2)], z0, mask=lane0)
        row = jnp.where(lane == 0, x0,
              jnp.where(lane == 1, y0,
              jnp.where(lane == 2, z0, jnp.float32(0.0))))
        cand_v[...] = row
        pltpu.sync_copy(cand_v, bc_sh.at[c, g])

    plsc.subcore_barrier()
    pltpu.sync_copy(bc_sh.at[c, g], bc_v)
    brow0 = bc_v[...]
    lastx = _splat(_lanek(brow0, 0))
    lasty = _splat(_lanek(brow0, 1))
    lastz = _splat(_lanek(brow0, 2))

    shard_base = j * NSH

    BIGIDX = jnp.float32(3.0e9)

    def step(t, carry):
        lx, ly, lz = carry

        @plsc.parallel_loop(0, CH, 1, unroll=UNROLL,
                            carry=(ninf16, _splat(rt0, jnp.int32)))
        def _dist(ci, st):
            bestv, bestc = st
            sl = pl.ds(ci * L, L)
            dx = x_v[sl] - lx
            dy = y_v[sl] - ly
            dz = z_v[sl] - lz
            dd = dx * dx + dy * dy + dz * dz
            nd = jnp.minimum(d_v[sl], dd)
            d_v[sl] = nd
            gt = nd > bestv
            bestv = jnp.where(gt, nd, bestv)
            bestc = jnp.where(gt, _splat(ci, jnp.int32), bestc)
            return bestv, bestc

        bestv, bestc = _dist

        m = jnp.max(bestv)
        gvec = shard_base + bestc * L + lane
        eq = bestv == _splat(m)
        lg = jnp.min(jnp.where(eq, gvec, jnp.int32(2**31 - 1)))
        p = _splat(lg - shard_base, jnp.int32)
        wx = plsc.load_gather(x_v, [p])
        wy = plsc.load_gather(y_v, [p])
        wz = plsc.load_gather(z_v, [p])

        rowv = jnp.where(lane == 0, _splat(m),
               jnp.where(lane == 1, wx,
               jnp.where(lane == 2, wy,
               jnp.where(lane == 3, wz,
               _splat(lg.astype(jnp.float32))))))
        cand_v[...] = rowv
        par = jnp.bitwise_and(t, 1)
        pltpu.sync_copy(cand_v, cand_sh.at[c, g, par, j])
        plsc.subcore_barrier()

        pltpu.sync_copy(cand_sh.at[c, g, par], cands_v)
        rowl = jnp.minimum(lane, WPB - 1)
        valid = lane < WPB
        vals = jnp.where(valid, plsc.load_gather(cands_v, [rowl, _idx(0)]), ninf16)
        gxs = plsc.load_gather(cands_v, [rowl, _idx(1)])
        gys = plsc.load_gather(cands_v, [rowl, _idx(2)])
        gzs = plsc.load_gather(cands_v, [rowl, _idx(3)])
        gis = jnp.where(valid, plsc.load_gather(cands_v, [rowl, _idx(4)]), _splat(BIGIDX))
        mm = jnp.max(vals)
        eqw = (vals == _splat(mm)) & valid
        gg = jnp.min(jnp.where(eqw, gis, _splat(BIGIDX)))
        eqg = eqw & (gis == _splat(gg))
        wxx = _splat(jnp.max(jnp.where(eqg, gxs, ninf16)))
        wyy = _splat(jnp.max(jnp.where(eqg, gys, ninf16)))
        wzz = _splat(jnp.max(jnp.where(eqg, gzs, ninf16)))

        @pl.when(j == 0)
        def _():
            tv = _splat(t, jnp.int32)
            plsc.store_scatter(samp_v, [tv, _idx(0)], wxx, mask=lane0)
            plsc.store_scatter(samp_v, [tv, _idx(1)], wyy, mask=lane0)
            plsc.store_scatter(samp_v, [tv, _idx(2)], wzz, mask=lane0)

        return wxx, wyy, wzz

    lax.fori_loop(1, Q, step, (lastx, lasty, lastz))

    @pl.when(j == 0)
    def _():
        pltpu.sync_copy(samp_v, samp_out.at[b])


_fps_call = pl.kernel(
    _fps_body,
    out_type=(
        jax.ShapeDtypeStruct((B, Q, D), jnp.float32),
        jax.ShapeDtypeStruct((B, Q, 3), jnp.float32),
    ),
    mesh=plsc.VectorSubcoreMesh(core_axis_name="c", subcore_axis_name="s",
                                num_cores=NC, num_subcores=NS),
    compiler_params=pltpu.CompilerParams(needs_layout_passes=False),
    scratch_types=[
        pltpu.VMEM((NPAD,), jnp.float32),
        pltpu.VMEM((NPAD,), jnp.float32),
        pltpu.VMEM((NPAD,), jnp.float32),
        pltpu.VMEM((NPAD,), jnp.float32),
        pltpu.VMEM((16, D), jnp.float32),
        pltpu.VMEM((Q, 3), jnp.float32),
        pltpu.VMEM((L,), jnp.float32),
        pltpu.VMEM((L,), jnp.float32),
        pltpu.VMEM((WPB, L), jnp.float32),
        pltpu.HBM((NC, NS // WPB, 2, WPB, L), jnp.float32),
        pltpu.HBM((NC, NS // WPB, L), jnp.float32),
    ],
)


def kernel(coordinates, query_weight):
    pts = coordinates.reshape(B, WPB, NSH, 3)
    pts = jnp.pad(pts, ((0, 0), (0, 0), (0, NPAD - NSH), (0, 0)))
    xs = pts[..., 0].reshape(B * WPB, NPAD)
    ys = pts[..., 1].reshape(B * WPB, NPAD)
    zs = pts[..., 2].reshape(B * WPB, NPAD)
    queries, sampled = _fps_call(xs, ys, zs, query_weight)
    return (queries, sampled)


# speedup vs baseline: 2.7560x; 1.0197x over previous
import jax
import jax.numpy as jnp
from jax import lax
from jax.experimental import pallas as pl
from jax.experimental.pallas import tpu as pltpu
from jax.experimental.pallas import tpu_sc as plsc

B = 4
N = 100000
Q = 128
D = 256
L = 16
NC = 2
NS = 16
WPB = 8
NSH = N // WPB
UNROLL = 8
CH = -(-NSH // (L * UNROLL)) * UNROLL
NPAD = CH * L


def _splat(x, dtype=None):
    v = jnp.broadcast_to(x, (L,))
    return v if dtype is None else v.astype(dtype)


def _fps_body(xs, ys, zs, qw, q_out, samp_out,
              x_v, y_v, z_v, d_v, qrows_v, samp_v, cand_v, bc_v, cands_v,
              cand_sh, bc_sh):
    c = lax.axis_index("c")
    s = lax.axis_index("s")
    g = s // WPB
    j = s % WPB
    b = c * (NS // WPB) + g
    r = b * WPB + j

    lane = lax.broadcasted_iota(jnp.int32, (L,), 0)
    ninf16 = jnp.full((L,), -jnp.inf, jnp.float32)
    lane0 = lane == 0
    rt0 = jnp.minimum(s, 0)

    def _idx(k):
        return _splat(rt0 + k, jnp.int32)

    def _lanek(v, k):
        return jnp.max(jnp.where(lane == k, v, ninf16))

    pltpu.sync_copy(qw.at[pl.ds(j * 16, 16)], qrows_v)
    pltpu.sync_copy(qrows_v, q_out.at[b, pl.ds(j * 16, 16)])

    pltpu.sync_copy(xs.at[r], x_v)
    pltpu.sync_copy(ys.at[r], y_v)
    pltpu.sync_copy(zs.at[r], z_v)

    def init_chunk(ci, _):
        gi = ci * L + lane
        d_v[pl.ds(ci * L, L)] = jnp.where(gi < NSH, jnp.inf, -jnp.inf).astype(jnp.float32)
        return 0

    lax.fori_loop(0, CH, init_chunk, 0)

    @pl.when(j == 0)
    def _():
        x0 = _splat(_lanek(x_v[pl.ds(0, L)], 0))
        y0 = _splat(_lanek(y_v[pl.ds(0, L)], 0))
        z0 = _splat(_lanek(z_v[pl.ds(0, L)], 0))
        plsc.store_scatter(samp_v, [_idx(0), _idx(0)], x0, mask=lane0)
        plsc.store_scatter(samp_v, [_idx(0), _idx(1)], y0, mask=lane0)
        plsc.store_scatter(samp_v, [_idx(0), _idx(2)], z0, mask=lane0)
        row = jnp.where(lane == 0, x0,
              jnp.where(lane == 1, y0,
              jnp.where(lane == 2, z0, jnp.float32(0.0))))
        cand_v[...] = row
        pltpu.sync_copy(cand_v, bc_sh.at[c, g])

    plsc.subcore_barrier()
    pltpu.sync_copy(bc_sh.at[c, g], bc_v)
    brow0 = bc_v[...]
    lastx = _splat(_lanek(brow0, 0))
    lasty = _splat(_lanek(brow0, 1))
    lastz = _splat(_lanek(brow0, 2))

    shard_base = j * NSH

    BIGIDX = jnp.float32(3.0e9)

    def step(t, carry):
        lx, ly, lz = carry

        @plsc.parallel_loop(0, CH, 1, unroll=UNROLL,
                            carry=(ninf16, _splat(rt0, jnp.int32)))
        def _dist(ci, st):
            bestv, bestc = st
            sl = pl.ds(ci * L, L)
            dx = x_v[sl] - lx
            dy = y_v[sl] - ly
            dz = z_v[sl] - lz
            dd = dx * dx + dy * dy + dz * dz
            nd = jnp.minimum(d_v[sl], dd)
            d_v[sl] = nd
            gt = nd > bestv
            bestv = jnp.where(gt, nd, bestv)
            bestc = jnp.where(gt, _splat(ci, jnp.int32), bestc)
            return bestv, bestc

        bestv, bestc = _dist

        m = jnp.max(bestv)
        gvec = shard_base + bestc * L + lane
        eq = bestv == _splat(m)
        lg = jnp.min(jnp.where(eq, gvec, jnp.int32(2**31 - 1)))
        p = _splat(lg - shard_base, jnp.int32)
        wx = plsc.load_gather(x_v, [p])
        wy = plsc.load_gather(y_v, [p])
        wz = plsc.load_gather(z_v, [p])

        rowv = jnp.where(lane == 0, _splat(m),
               jnp.where(lane == 1, wx,
               jnp.where(lane == 2, wy,
               jnp.where(lane == 3, wz,
               _splat(lg.astype(jnp.float32))))))
        cand_v[...] = rowv
        par = jnp.bitwise_and(t, 1)
        pltpu.sync_copy(cand_v, cand_sh.at[c, g, par, j])
        plsc.subcore_barrier()

        pltpu.sync_copy(cand_sh.at[c, g, par], cands_v)
        rowl = jnp.minimum(lane, WPB - 1)
        valid = lane < WPB
        vals = jnp.where(valid, plsc.load_gather(cands_v, [rowl, _idx(0)]), ninf16)
        gxs = plsc.load_gather(cands_v, [rowl, _idx(1)])
        gys = plsc.load_gather(cands_v, [rowl, _idx(2)])
        gzs = plsc.load_gather(cands_v, [rowl, _idx(3)])
        gis = jnp.where(valid, plsc.load_gather(cands_v, [rowl, _idx(4)]), _splat(BIGIDX))
        mm = jnp.max(vals)
        eqw = (vals == _splat(mm)) & valid
        gg = jnp.min(jnp.where(eqw, gis, _splat(BIGIDX)))
        eqg = eqw & (gis == _splat(gg))
        wxx = _splat(jnp.max(jnp.where(eqg, gxs, ninf16)))
        wyy = _splat(jnp.max(jnp.where(eqg, gys, ninf16)))
        wzz = _splat(jnp.max(jnp.where(eqg, gzs, ninf16)))

        @pl.when(j == 0)
        def _():
            tv = _splat(t, jnp.int32)
            plsc.store_scatter(samp_v, [tv, _idx(0)], wxx, mask=lane0)
            plsc.store_scatter(samp_v, [tv, _idx(1)], wyy, mask=lane0)
            plsc.store_scatter(samp_v, [tv, _idx(2)], wzz, mask=lane0)

        return wxx, wyy, wzz

    lax.fori_loop(1, Q, step, (lastx, lasty, lastz))

    @pl.when(j == 0)
    def _():
        pltpu.sync_copy(samp_v, samp_out.at[b])


_fps_call = pl.kernel(
    _fps_body,
    out_type=(
        jax.ShapeDtypeStruct((B, Q, D), jnp.float32),
        jax.ShapeDtypeStruct((B, Q, 3), jnp.float32),
    ),
    mesh=plsc.VectorSubcoreMesh(core_axis_name="c", subcore_axis_name="s",
                                num_cores=NC, num_subcores=NS),
    compiler_params=pltpu.CompilerParams(needs_layout_passes=False),
    scratch_types=[
        pltpu.VMEM((NPAD,), jnp.float32),
        pltpu.VMEM((NPAD,), jnp.float32),
        pltpu.VMEM((NPAD,), jnp.float32),
        pltpu.VMEM((NPAD,), jnp.float32),
        pltpu.VMEM((16, D), jnp.float32),
        pltpu.VMEM((Q, 3), jnp.float32),
        pltpu.VMEM((L,), jnp.float32),
        pltpu.VMEM((L,), jnp.float32),
        pltpu.VMEM((WPB, L), jnp.float32),
        pltpu.HBM((NC, NS // WPB, 2, WPB, L), jnp.float32),
        pltpu.HBM((NC, NS // WPB, L), jnp.float32),
    ],
)


def kernel(coordinates, query_weight):
    pts = coordinates.reshape(B, WPB, NSH, 3)
    pts = jnp.pad(pts, ((0, 0), (0, 0), (0, NPAD - NSH), (0, 0)))
    xs = pts[..., 0].reshape(B * WPB, NPAD)
    ys = pts[..., 1].reshape(B * WPB, NPAD)
    zs = pts[..., 2].reshape(B * WPB, NPAD)
    queries, sampled = _fps_call(xs, ys, zs, query_weight)
    return (queries, sampled)


# speedup vs baseline: 2.7972x; 1.0149x over previous
import jax
import jax.numpy as jnp
from jax import lax
from jax.experimental import pallas as pl
from jax.experimental.pallas import tpu as pltpu
from jax.experimental.pallas import tpu_sc as plsc

B = 4
N = 100000
Q = 128
D = 256
L = 16
NC = 2
NS = 16
WPB = 8
NSH = N // WPB
UNROLL = 16
CH = -(-NSH // (L * UNROLL)) * UNROLL
NPAD = CH * L


def _splat(x, dtype=None):
    v = jnp.broadcast_to(x, (L,))
    return v if dtype is None else v.astype(dtype)


def _fps_body(xs, ys, zs, qw, q_out, samp_out,
              x_v, y_v, z_v, d_v, qrows_v, samp_v, cand_v, bc_v, cands_v,
              cand_sh, bc_sh):
    c = lax.axis_index("c")
    s = lax.axis_index("s")
    g = s // WPB
    j = s % WPB
    b = c * (NS // WPB) + g
    r = b * WPB + j

    lane = lax.broadcasted_iota(jnp.int32, (L,), 0)
    ninf16 = jnp.full((L,), -jnp.inf, jnp.float32)
    lane0 = lane == 0
    rt0 = jnp.minimum(s, 0)

    def _idx(k):
        return _splat(rt0 + k, jnp.int32)

    def _lanek(v, k):
        return jnp.max(jnp.where(lane == k, v, ninf16))

    pltpu.sync_copy(qw.at[pl.ds(j * 16, 16)], qrows_v)
    pltpu.sync_copy(qrows_v, q_out.at[b, pl.ds(j * 16, 16)])

    pltpu.sync_copy(xs.at[r], x_v)
    pltpu.sync_copy(ys.at[r], y_v)
    pltpu.sync_copy(zs.at[r], z_v)

    def init_chunk(ci, _):
        gi = ci * L + lane
        d_v[pl.ds(ci * L, L)] = jnp.where(gi < NSH, jnp.inf, -jnp.inf).astype(jnp.float32)
        return 0

    lax.fori_loop(0, CH, init_chunk, 0)

    @pl.when(j == 0)
    def _():
        x0 = _splat(_lanek(x_v[pl.ds(0, L)], 0))
        y0 = _splat(_lanek(y_v[pl.ds(0, L)], 0))
        z0 = _splat(_lanek(z_v[pl.ds(0, L)], 0))
        plsc.store_scatter(samp_v, [_idx(0), _idx(0)], x0, mask=lane0)
        plsc.store_scatter(samp_v, [_idx(0), _idx(1)], y0, mask=lane0)
        plsc.store_scatter(samp_v, [_idx(0), _idx(2)], z0, mask=lane0)
        row = jnp.where(lane == 0, x0,
              jnp.where(lane == 1, y0,
              jnp.where(lane == 2, z0, jnp.float32(0.0))))
        cand_v[...] = row
        pltpu.sync_copy(cand_v, bc_sh.at[c, g])

    plsc.subcore_barrier()
    pltpu.sync_copy(bc_sh.at[c, g], bc_v)
    brow0 = bc_v[...]
    lastx = _splat(_lanek(brow0, 0))
    lasty = _splat(_lanek(brow0, 1))
    lastz = _splat(_lanek(brow0, 2))

    shard_base = j * NSH

    BIGIDX = jnp.float32(3.0e9)

    def step(t, carry):
        lx, ly, lz = carry

        @plsc.parallel_loop(0, CH, 1, unroll=UNROLL,
                            carry=(ninf16, _splat(rt0, jnp.int32)))
        def _dist(ci, st):
            bestv, bestc = st
            sl = pl.ds(ci * L, L)
            dx = x_v[sl] - lx
            dy = y_v[sl] - ly
            dz = z_v[sl] - lz
            dd = dx * dx + dy * dy + dz * dz
            nd = jnp.minimum(d_v[sl], dd)
            d_v[sl] = nd
            gt = nd > bestv
            bestv = jnp.where(gt, nd, bestv)
            bestc = jnp.where(gt, _splat(ci, jnp.int32), bestc)
            return bestv, bestc

        bestv, bestc = _dist

        m = jnp.max(bestv)
        gvec = shard_base + bestc * L + lane
        eq = bestv == _splat(m)
        lg = jnp.min(jnp.where(eq, gvec, jnp.int32(2**31 - 1)))
        p = _splat(lg - shard_base, jnp.int32)
        wx = plsc.load_gather(x_v, [p])
        wy = plsc.load_gather(y_v, [p])
        wz = plsc.load_gather(z_v, [p])

        rowv = jnp.where(lane == 0, _splat(m),
               jnp.where(lane == 1, wx,
               jnp.where(lane == 2, wy,
               jnp.where(lane == 3, wz,
               _splat(lg.astype(jnp.float32))))))
        cand_v[...] = rowv
        par = jnp.bitwise_and(t, 1)
        pltpu.sync_copy(cand_v, cand_sh.at[c, g, par, j])
        plsc.subcore_barrier()

        pltpu.sync_copy(cand_sh.at[c, g, par], cands_v)
        rowl = jnp.minimum(lane, WPB - 1)
        valid = lane < WPB
        vals = jnp.where(valid, plsc.load_gather(cands_v, [rowl, _idx(0)]), ninf16)
        gxs = plsc.load_gather(cands_v, [rowl, _idx(1)])
        gys = plsc.load_gather(cands_v, [rowl, _idx(2)])
        gzs = plsc.load_gather(cands_v, [rowl, _idx(3)])
        gis = jnp.where(valid, plsc.load_gather(cands_v, [rowl, _idx(4)]), _splat(BIGIDX))
        mm = jnp.max(vals)
        eqw = (vals == _splat(mm)) & valid
        gg = jnp.min(jnp.where(eqw, gis, _splat(BIGIDX)))
        eqg = eqw & (gis == _splat(gg))
        wxx = _splat(jnp.max(jnp.where(eqg, gxs, ninf16)))
        wyy = _splat(jnp.max(jnp.where(eqg, gys, ninf16)))
        wzz = _splat(jnp.max(jnp.where(eqg, gzs, ninf16)))

        @pl.when(j == 0)
        def _():
            tv = _splat(t, jnp.int32)
            plsc.store_scatter(samp_v, [tv, _idx(0)], wxx, mask=lane0)
            plsc.store_scatter(samp_v, [tv, _idx(1)], wyy, mask=lane0)
            plsc.store_scatter(samp_v, [tv, _idx(2)], wzz, mask=lane0)

        return wxx, wyy, wzz

    lax.fori_loop(1, Q, step, (lastx, lasty, lastz))

    @pl.when(j == 0)
    def _():
        pltpu.sync_copy(samp_v, samp_out.at[b])


_fps_call = pl.kernel(
    _fps_body,
    out_type=(
        jax.ShapeDtypeStruct((B, Q, D), jnp.float32),
        jax.ShapeDtypeStruct((B, Q, 3), jnp.float32),
    ),
    mesh=plsc.VectorSubcoreMesh(core_axis_name="c", subcore_axis_name="s",
                                num_cores=NC, num_subcores=NS),
    compiler_params=pltpu.CompilerParams(needs_layout_passes=False),
    scratch_types=[
        pltpu.VMEM((NPAD,), jnp.float32),
        pltpu.VMEM((NPAD,), jnp.float32),
        pltpu.VMEM((NPAD,), jnp.float32),
        pltpu.VMEM((NPAD,), jnp.float32),
        pltpu.VMEM((16, D), jnp.float32),
        pltpu.VMEM((Q, 3), jnp.float32),
        pltpu.VMEM((L,), jnp.float32),
        pltpu.VMEM((L,), jnp.float32),
        pltpu.VMEM((WPB, L), jnp.float32),
        pltpu.HBM((NC, NS // WPB, 2, WPB, L), jnp.float32),
        pltpu.HBM((NC, NS // WPB, L), jnp.float32),
    ],
)


def kernel(coordinates, query_weight):
    pts = coordinates.reshape(B, WPB, NSH, 3)
    pts = jnp.pad(pts, ((0, 0), (0, 0), (0, NPAD - NSH), (0, 0)))
    xs = pts[..., 0].reshape(B * WPB, NPAD)
    ys = pts[..., 1].reshape(B * WPB, NPAD)
    zs = pts[..., 2].reshape(B * WPB, NPAD)
    queries, sampled = _fps_call(xs, ys, zs, query_weight)
    return (queries, sampled)
